# baseline (speedup 1.0000x reference)
import functools

import jax
import jax.numpy as jnp
from jax import lax
from jax.experimental import pallas as pl
from jax.experimental.pallas import tpu as pltpu
from jax.experimental.pallas import tpu_sc as plsc

N = 10000
E = 320000
NC = 2
NS = 16
NW = NC * NS
EP = E // NW
CHUNK = 80
NCHUNK = EP // CHUNK
RPT = 624
ZROWS = 208


def _sc_degree(dst):
    mesh = plsc.VectorSubcoreMesh(core_axis_name="c", subcore_axis_name="s")

    @functools.partial(
        pl.kernel,
        out_type=jax.ShapeDtypeStruct((NC * N,), jnp.float32),
        mesh=mesh,
        scratch_types=(
            [pltpu.VMEM((80,), jnp.int32)] * 5
            + [pltpu.VMEM((80,), jnp.float32),
               pltpu.VMEM((RPT,), jnp.float32),
               pltpu.VMEM_SHARED((N,), jnp.float32)]
            + [pltpu.SemaphoreType.DMA] * 10
        ),
    )
    def k(dst_hbm, out_hbm, *sc):
        idx = sc[0:5]
        ones_v = sc[5]
        zbuf = sc[6]
        acc_sh = sc[7]
        isem = sc[8:13]
        ssem = sc[13:18]
        c = lax.axis_index("c")
        s = lax.axis_index("s")
        wid = c * NS + s
        for i in range(80 // 16):
            ones_v[pl.ds(i * 16, 16)] = jnp.ones((16,), jnp.float32)

        @pl.loop(0, RPT // 16)
        def _(i):
            zbuf[pl.ds(i * 16, 16)] = jnp.zeros((16,), jnp.float32)

        pltpu.sync_copy(zbuf, acc_sh.at[pl.ds(s * RPT, RPT)])

        @pl.when(s == NS - 1)
        def _():
            pltpu.sync_copy(zbuf.at[pl.ds(0, 16)],
                            acc_sh.at[pl.ds(NS * RPT, 16)])

        plsc.subcore_barrier()

        def fire_idx(g, b):
            base = wid * EP + (g * 5 + b) * 80
            pltpu.async_copy(dst_hbm.at[pl.ds(base, 80)], idx[b], isem[b])

        for b in range(5):
            fire_idx(0, b)

        def wait_scatter(b):
            pltpu.make_async_copy(ones_v, acc_sh.at[idx[b]], ssem[b]).wait()

        @pl.loop(0, NCHUNK // 5)
        def _(g):
            for b in range(5):
                pltpu.make_async_copy(dst_hbm.at[pl.ds(0, 80)], idx[b],
                                      isem[b]).wait()
                pltpu.async_copy(ones_v, acc_sh.at[idx[b]], ssem[b], add=True)
            for b in range(5):
                def refill(b=b):
                    wait_scatter(b)
                    fire_idx(g + 1, b)
                pl.when(g < NCHUNK // 5 - 1)(refill)

        for b in range(5):
            wait_scatter(b)

        plsc.subcore_barrier()
        pltpu.sync_copy(acc_sh.at[pl.ds(s * RPT, RPT)], zbuf)
        pltpu.sync_copy(zbuf, out_hbm.at[pl.ds(c * N + s * RPT, RPT)])

        @pl.when(s == NS - 1)
        def _():
            pltpu.sync_copy(acc_sh.at[pl.ds(NS * RPT, 16)],
                            ones_v.at[pl.ds(0, 16)])
            pltpu.sync_copy(ones_v.at[pl.ds(0, 16)],
                            out_hbm.at[pl.ds(c * N + NS * RPT, 16)])

    return k(dst)


CH = 80
NB = 4
SP = 104
NG1 = SP // NB
NG2 = 5


def _sc_agg(src, dst, xws):
    d = xws.shape[1]
    mesh = plsc.VectorSubcoreMesh(core_axis_name="c", subcore_axis_name="s")

    @functools.partial(
        pl.kernel,
        out_type=jax.ShapeDtypeStruct((NC, N, d), jnp.float32),
        mesh=mesh,
        scratch_types=(
            [pltpu.VMEM((CH,), jnp.int32)] * (2 * NB)
            + [pltpu.VMEM((CH, d), jnp.float32)] * NB
            + [pltpu.VMEM((SP * CH,), jnp.int32)]
            + [pltpu.VMEM_SHARED((N, d), jnp.float32)]
            + [pltpu.SemaphoreType.DMA] * (3 * NB + 1)
        ),
    )
    def k(src_hbm, dst_hbm, xws_hbm, out_hbm, *sc):
        sidx = sc[0:NB]
        didx = sc[NB:2 * NB]
        rows = sc[2 * NB:3 * NB]
        sidx_all = sc[3 * NB]
        acc_sh = sc[3 * NB + 1]
        isem = sc[3 * NB + 2:4 * NB + 2]
        gsem = sc[4 * NB + 2:5 * NB + 2]
        ssem = sc[5 * NB + 2:6 * NB + 2]
        asem = sc[6 * NB + 2]
        c = lax.axis_index("c")
        s = lax.axis_index("s")
        wid = c * NS + s

        def fire_idx(g, b):
            base = wid * EP + (g * NB + b) * CH
            pltpu.async_copy(src_hbm.at[pl.ds(base, CH)], sidx[b], isem[b])
            pltpu.async_copy(dst_hbm.at[pl.ds(base, CH)], didx[b], isem[b])

        def wait_idx(b):
            pltpu.make_async_copy(src_hbm.at[pl.ds(0, CH)], sidx[b],
                                  isem[b]).wait()
            pltpu.make_async_copy(src_hbm.at[pl.ds(0, CH)], didx[b],
                                  isem[b]).wait()

        def fire_idxd(g, b):
            base = wid * EP + (g * NB + b) * CH
            pltpu.async_copy(dst_hbm.at[pl.ds(base, CH)], didx[b], isem[b])

        def wait_idxd(b):
            pltpu.make_async_copy(src_hbm.at[pl.ds(0, CH)], didx[b],
                                  isem[b]).wait()

        def fire_gather(b):
            pltpu.async_copy(xws_hbm.at[sidx[b]], rows[b], gsem[b])

        def wait_gather(b):
            pltpu.make_async_copy(xws_hbm.at[sidx[b]], rows[b],
                                  gsem[b]).wait()

        def fire_gather_pre(ch, b):
            pltpu.async_copy(
                xws_hbm.at[sidx_all.at[pl.ds(ch * CH, CH)]], rows[b],
                gsem[b])

        def wait_gather_pre(ch, b):
            pltpu.make_async_copy(
                xws_hbm.at[sidx_all.at[pl.ds(ch * CH, CH)]], rows[b],
                gsem[b]).wait()

        def fire_scatter(b):
            pltpu.async_copy(rows[b], acc_sh.at[didx[b]], ssem[b], add=True)

        def wait_scatter(b):
            pltpu.make_async_copy(rows[b], acc_sh.at[didx[b]],
                                  ssem[b]).wait()

        @pl.loop(0, CH)
        def _(i):
            for j in range(d // 16):
                rows[0][i, pl.ds(j * 16, 16)] = jnp.zeros((16,), jnp.float32)

        zp = [(i * CH, CH) for i in range(7)] + [(560, 64)]
        for i, (r0, sz) in enumerate(zp):
            sem = (gsem + ssem)[i]
            pltpu.async_copy(rows[0].at[pl.ds(0, sz)],
                             acc_sh.at[pl.ds(s * RPT + r0, sz)], sem)

        @pl.when(s == NS - 1)
        def _():
            pltpu.async_copy(rows[0].at[pl.ds(0, 16)],
                             acc_sh.at[pl.ds(NS * RPT, 16)], isem[0])
            pltpu.make_async_copy(rows[0].at[pl.ds(0, 16)],
                                  acc_sh.at[pl.ds(NS * RPT, 16)],
                                  isem[0]).wait()

        for i, (r0, sz) in enumerate(zp):
            sem = (gsem + ssem)[i]
            pltpu.make_async_copy(rows[0].at[pl.ds(0, sz)],
                                  acc_sh.at[pl.ds(s * RPT + r0, sz)],
                                  sem).wait()

        plsc.subcore_barrier()

        pltpu.async_copy(src_hbm.at[pl.ds(wid * EP, SP * CH)], sidx_all,
                         asem)
        for b in range(NB):
            fire_idxd(0, b)
        pltpu.make_async_copy(src_hbm.at[pl.ds(0, SP * CH)], sidx_all,
                              asem).wait()

        @pl.loop(0, NG1)
        def _(g):
            for b in range(NB):
                wait_idxd(b)
                fire_gather_pre(g * NB + b, b)
            for b in range(NB):
                wait_gather_pre(g * NB + b, b)
                fire_scatter(b)
            for b in range(NB):
                def refill(b=b):
                    wait_scatter(b)
                    pl.when(g < NG1 - 1)(lambda: fire_idxd(g + 1, b))
                    pl.when(g == NG1 - 1)(lambda: fire_idx(NG1, b))
                refill()

        for g2 in range(NG1, NG1 + NG2):
            for b in range(NB):
                wait_idx(b)
                fire_gather(b)
            for b in range(NB):
                wait_gather(b)
                fire_scatter(b)
            for b in range(NB):
                wait_scatter(b)
                if g2 + 1 < NG1 + NG2:
                    fire_idx(g2 + 1, b)

        fire_idx(NG1 + NG2, 0)
        wait_idx(0)
        fire_gather(0)
        wait_gather(0)
        fire_scatter(0)
        wait_scatter(0)

        plsc.subcore_barrier()

        wp = [(i * CH, CH) for i in range(7)] + [(560, 64)]

        def fire_load(i):
            r0, sz = wp[i]
            pltpu.async_copy(acc_sh.at[pl.ds(s * RPT + r0, sz)],
                             rows[i % NB].at[pl.ds(0, sz)], gsem[i % NB])

        def wait_load(i):
            r0, sz = wp[i]
            pltpu.make_async_copy(acc_sh.at[pl.ds(s * RPT + r0, sz)],
                                  rows[i % NB].at[pl.ds(0, sz)],
                                  gsem[i % NB]).wait()

        def fire_store(i):
            r0, sz = wp[i]
            pltpu.async_copy(rows[i % NB].at[pl.ds(0, sz)],
                             out_hbm.at[c, pl.ds(s * RPT + r0, sz)],
                             ssem[i % NB])

        def wait_store(i):
            r0, sz = wp[i]
            pltpu.make_async_copy(rows[i % NB].at[pl.ds(0, sz)],
                                  out_hbm.at[c, pl.ds(s * RPT + r0, sz)],
                                  ssem[i % NB]).wait()

        for i in range(NB):
            fire_load(i)
        for i in range(len(wp)):
            wait_load(i)
            fire_store(i)
            if i + NB < len(wp):
                wait_store(i)
                fire_load(i + NB)
        for i in range(len(wp) - NB, len(wp)):
            wait_store(i)

        @pl.when(s == NS - 1)
        def _():
            pltpu.sync_copy(acc_sh.at[pl.ds(NS * RPT, 16)],
                            rows[0].at[pl.ds(0, 16)])
            pltpu.sync_copy(rows[0].at[pl.ds(0, 16)],
                            out_hbm.at[c, pl.ds(NS * RPT, 16)])

    return k(src, dst, xws)


_R = 1000


def _tc_mm1(x, w1):
    def body(x_ref, w_ref, out_ref):
        out_ref[...] = jnp.dot(x_ref[...], w_ref[...],
                               preferred_element_type=jnp.float32)

    return pl.pallas_call(
        body,
        grid=(N // _R,),
        in_specs=[
            pl.BlockSpec((_R, 128), lambda i: (i, 0)),
            pl.BlockSpec((128, 128), lambda i: (0, 0)),
        ],
        out_specs=pl.BlockSpec((_R, 128), lambda i: (i, 0)),
        out_shape=jax.ShapeDtypeStruct((N, 128), jnp.float32),
    )(x, w1)


def _tc_scale(degp_t, xw1):
    def body(degp_ref, xw_ref, dinv_ref, xws_ref):
        deg = degp_ref[:, 0:1] + degp_ref[:, 1:2] + 1.0
        dinv = lax.rsqrt(deg)
        dinv_ref[...] = dinv
        xws_ref[...] = xw_ref[...] * dinv

    return pl.pallas_call(
        body,
        grid=(N // _R,),
        in_specs=[
            pl.BlockSpec((_R, NC), lambda i: (i, 0)),
            pl.BlockSpec((_R, 128), lambda i: (i, 0)),
        ],
        out_specs=[
            pl.BlockSpec((_R, 1), lambda i: (i, 0)),
            pl.BlockSpec((_R, 128), lambda i: (i, 0)),
        ],
        out_shape=[
            jax.ShapeDtypeStruct((N, 1), jnp.float32),
            jax.ShapeDtypeStruct((N, 128), jnp.float32),
        ],
    )(degp_t, xw1)


def _tc_mid(acc1, xw1s, dinv, b1, gamma, beta, w2):
    def body(acc_ref, xws_ref, dinv_ref, b_ref, g_ref, be_ref, w_ref, out_ref):
        dinv = dinv_ref[...]
        h = (acc_ref[0] + acc_ref[1] + xws_ref[...]) * dinv + b_ref[...]
        mu = jnp.mean(h, axis=-1, keepdims=True)
        var = jnp.mean((h - mu) ** 2, axis=-1, keepdims=True)
        h = (h - mu) / jnp.sqrt(var + 1e-5) * g_ref[...] + be_ref[...]
        h = jnp.maximum(h, 0.0)
        res = jnp.dot(h, w_ref[...],
                      preferred_element_type=jnp.float32) * dinv
        out_ref[...] = jnp.concatenate(
            [res, jnp.zeros((_R, 64), jnp.float32)], axis=1)

    return pl.pallas_call(
        body,
        grid=(N // _R,),
        in_specs=[
            pl.BlockSpec((NC, _R, 128), lambda i: (0, i, 0)),
            pl.BlockSpec((_R, 128), lambda i: (i, 0)),
            pl.BlockSpec((_R, 1), lambda i: (i, 0)),
            pl.BlockSpec((1, 128), lambda i: (0, 0)),
            pl.BlockSpec((1, 128), lambda i: (0, 0)),
            pl.BlockSpec((1, 128), lambda i: (0, 0)),
            pl.BlockSpec((128, 64), lambda i: (0, 0)),
        ],
        out_specs=pl.BlockSpec((_R, 128), lambda i: (i, 0)),
        out_shape=jax.ShapeDtypeStruct((N, 128), jnp.float32),
    )(acc1, xw1s, dinv, b1, gamma, beta, w2)


def _tc_feat(acc2, xw2s, dinv, b2, wdec):
    def body(acc_ref, xws_ref, dinv_ref, b_ref, wdec_ref, feat_ref, xw3_ref):
        dinv = dinv_ref[...]
        feat = ((acc_ref[0, :, :64] + acc_ref[1, :, :64] + xws_ref[:, :64])
                * dinv + b_ref[...])
        feat_ref[...] = feat
        xw3_ref[...] = jnp.dot(feat, wdec_ref[...],
                               preferred_element_type=jnp.float32) * dinv

    return pl.pallas_call(
        body,
        grid=(N // _R,),
        in_specs=[
            pl.BlockSpec((NC, _R, 128), lambda i: (0, i, 0)),
            pl.BlockSpec((_R, 128), lambda i: (i, 0)),
            pl.BlockSpec((_R, 1), lambda i: (i, 0)),
            pl.BlockSpec((1, 64), lambda i: (0, 0)),
            pl.BlockSpec((64, 128), lambda i: (0, 0)),
        ],
        out_specs=[
            pl.BlockSpec((_R, 64), lambda i: (i, 0)),
            pl.BlockSpec((_R, 128), lambda i: (i, 0)),
        ],
        out_shape=[
            jax.ShapeDtypeStruct((N, 64), jnp.float32),
            jax.ShapeDtypeStruct((N, 128), jnp.float32),
        ],
    )(acc2, xw2s, dinv, b2, wdec)


def _tc_margin(feat, wlab, wdom):
    def body(feat_ref, wlab_ref, wdom_ref, lab_ref, dom_ref):
        feat = feat_ref[...]
        fn = feat / jnp.maximum(
            jnp.sqrt(jnp.sum(feat * feat, axis=1, keepdims=True)), 1e-12)
        wlab = wlab_ref[...]
        wlab = wlab / jnp.maximum(
            jnp.sqrt(jnp.sum(wlab * wlab, axis=0, keepdims=True)), 1e-12)
        lab_ref[...] = 5.0 * jnp.dot(fn, wlab,
                                     preferred_element_type=jnp.float32)
        wdom = wdom_ref[...]
        wdom = wdom / jnp.maximum(
            jnp.sqrt(jnp.sum(wdom * wdom, axis=0, keepdims=True)), 1e-12)
        dom_ref[...] = 5.0 * jnp.dot(fn, wdom,
                                     preferred_element_type=jnp.float32)

    return pl.pallas_call(
        body,
        grid=(N // _R,),
        in_specs=[
            pl.BlockSpec((_R, 64), lambda i: (i, 0)),
            pl.BlockSpec((64, 64), lambda i: (0, 0)),
            pl.BlockSpec((64, 4), lambda i: (0, 0)),
        ],
        out_specs=[
            pl.BlockSpec((_R, 64), lambda i: (i, 0)),
            pl.BlockSpec((_R, 4), lambda i: (i, 0)),
        ],
        out_shape=[
            jax.ShapeDtypeStruct((N, 64), jnp.float32),
            jax.ShapeDtypeStruct((N, 4), jnp.float32),
        ],
    )(feat, wlab, wdom)


def _tc_final(acc3, xw3s, dinv, bdec):
    def body(acc_ref, xws_ref, dinv_ref, b_ref, out_ref):
        out_ref[...] = ((acc_ref[0] + acc_ref[1] + xws_ref[...])
                        * dinv_ref[...] + b_ref[...])

    return pl.pallas_call(
        body,
        grid=(N // _R,),
        in_specs=[
            pl.BlockSpec((NC, _R, 128), lambda i: (0, i, 0)),
            pl.BlockSpec((_R, 128), lambda i: (i, 0)),
            pl.BlockSpec((_R, 1), lambda i: (i, 0)),
            pl.BlockSpec((1, 128), lambda i: (0, 0)),
        ],
        out_specs=pl.BlockSpec((_R, 128), lambda i: (i, 0)),
        out_shape=jax.ShapeDtypeStruct((N, 128), jnp.float32),
    )(acc3, xw3s, dinv, bdec)


def kernel(x, edge_index, W1, b1, gamma, beta, W2, b2, Wdec, bdec, Wdom, Wlab):
    src = edge_index[0]
    dst = edge_index[1]
    degp = _sc_degree(dst)
    xw1 = _tc_mm1(x, W1)
    dinv, xw1s = _tc_scale(degp.reshape(NC, N).T, xw1)
    acc1 = _sc_agg(src, dst, xw1s)
    xw2s = _tc_mid(acc1, xw1s, dinv, b1.reshape(1, -1), gamma.reshape(1, -1),
                   beta.reshape(1, -1), W2)
    acc2 = _sc_agg(src, dst, xw2s)
    feat, xw3s = _tc_feat(acc2, xw2s, dinv, b2.reshape(1, -1), Wdec)
    acc3 = _sc_agg(src, dst, xw3s)
    label_pred, domain_pred = _tc_margin(feat, Wlab, Wdom)
    recon = _tc_final(acc3, xw3s, dinv, bdec.reshape(1, -1))
    return (feat, domain_pred, recon, label_pred)


# speedup vs baseline: 1.0047x; 1.0047x over previous
import functools

import jax
import jax.numpy as jnp
from jax import lax
from jax.experimental import pallas as pl
from jax.experimental.pallas import tpu as pltpu
from jax.experimental.pallas import tpu_sc as plsc

N = 10000
E = 320000
NC = 2
NS = 16
NW = NC * NS
EP = E // NW
CHUNK = 80
NCHUNK = EP // CHUNK
RPT = 624
ZROWS = 208


def _sc_degree(dst):
    mesh = plsc.VectorSubcoreMesh(core_axis_name="c", subcore_axis_name="s")

    @functools.partial(
        pl.kernel,
        out_type=jax.ShapeDtypeStruct((NC * N,), jnp.float32),
        mesh=mesh,
        scratch_types=(
            [pltpu.VMEM((80,), jnp.int32)] * 5
            + [pltpu.VMEM((80,), jnp.float32),
               pltpu.VMEM((RPT,), jnp.float32),
               pltpu.VMEM_SHARED((N,), jnp.float32)]
            + [pltpu.SemaphoreType.DMA] * 10
        ),
    )
    def k(dst_hbm, out_hbm, *sc):
        idx = sc[0:5]
        ones_v = sc[5]
        zbuf = sc[6]
        acc_sh = sc[7]
        isem = sc[8:13]
        ssem = sc[13:18]
        c = lax.axis_index("c")
        s = lax.axis_index("s")
        wid = c * NS + s
        for i in range(80 // 16):
            ones_v[pl.ds(i * 16, 16)] = jnp.ones((16,), jnp.float32)

        @pl.loop(0, RPT // 16)
        def _(i):
            zbuf[pl.ds(i * 16, 16)] = jnp.zeros((16,), jnp.float32)

        pltpu.sync_copy(zbuf, acc_sh.at[pl.ds(s * RPT, RPT)])

        @pl.when(s == NS - 1)
        def _():
            pltpu.sync_copy(zbuf.at[pl.ds(0, 16)],
                            acc_sh.at[pl.ds(NS * RPT, 16)])

        plsc.subcore_barrier()

        def fire_idx(g, b):
            base = wid * EP + (g * 5 + b) * 80
            pltpu.async_copy(dst_hbm.at[pl.ds(base, 80)], idx[b], isem[b])

        for b in range(5):
            fire_idx(0, b)

        def wait_scatter(b):
            pltpu.make_async_copy(ones_v, acc_sh.at[idx[b]], ssem[b]).wait()

        @pl.loop(0, NCHUNK // 5)
        def _(g):
            for b in range(5):
                pltpu.make_async_copy(dst_hbm.at[pl.ds(0, 80)], idx[b],
                                      isem[b]).wait()
                pltpu.async_copy(ones_v, acc_sh.at[idx[b]], ssem[b], add=True)
            for b in range(5):
                def refill(b=b):
                    wait_scatter(b)
                    fire_idx(g + 1, b)
                pl.when(g < NCHUNK // 5 - 1)(refill)

        for b in range(5):
            wait_scatter(b)

        plsc.subcore_barrier()
        pltpu.sync_copy(acc_sh.at[pl.ds(s * RPT, RPT)], zbuf)
        pltpu.sync_copy(zbuf, out_hbm.at[pl.ds(c * N + s * RPT, RPT)])

        @pl.when(s == NS - 1)
        def _():
            pltpu.sync_copy(acc_sh.at[pl.ds(NS * RPT, 16)],
                            ones_v.at[pl.ds(0, 16)])
            pltpu.sync_copy(ones_v.at[pl.ds(0, 16)],
                            out_hbm.at[pl.ds(c * N + NS * RPT, 16)])

    return k(dst)


CH = 80
NB = 4
NGRP = 31


def _sc_agg(src, dst, xws):
    d = xws.shape[1]
    mesh = plsc.VectorSubcoreMesh(core_axis_name="c", subcore_axis_name="s")

    @functools.partial(
        pl.kernel,
        out_type=jax.ShapeDtypeStruct((NC, N, d), jnp.float32),
        mesh=mesh,
        scratch_types=(
            [pltpu.VMEM((CH,), jnp.int32)] * (2 * NB)
            + [pltpu.VMEM((CH, d), jnp.float32)] * NB
            + [pltpu.VMEM_SHARED((N, d), jnp.float32)]
            + [pltpu.SemaphoreType.DMA] * (3 * NB)
        ),
    )
    def k(src_hbm, dst_hbm, xws_hbm, out_hbm, *sc):
        sidx = sc[0:NB]
        didx = sc[NB:2 * NB]
        rows = sc[2 * NB:3 * NB]
        acc_sh = sc[3 * NB]
        isem = sc[3 * NB + 1:4 * NB + 1]
        gsem = sc[4 * NB + 1:5 * NB + 1]
        ssem = sc[5 * NB + 1:6 * NB + 1]
        c = lax.axis_index("c")
        s = lax.axis_index("s")
        wid = c * NS + s

        def fire_idx(g, b):
            base = wid * EP + (g * NB + b) * CH
            pltpu.async_copy(src_hbm.at[pl.ds(base, CH)], sidx[b], isem[b])
            pltpu.async_copy(dst_hbm.at[pl.ds(base, CH)], didx[b], isem[b])

        def wait_idx(b):
            pltpu.make_async_copy(src_hbm.at[pl.ds(0, CH)], sidx[b],
                                  isem[b]).wait()
            pltpu.make_async_copy(src_hbm.at[pl.ds(0, CH)], didx[b],
                                  isem[b]).wait()

        def fire_gather(b):
            pltpu.async_copy(xws_hbm.at[sidx[b]], rows[b], gsem[b])

        def wait_gather(b):
            pltpu.make_async_copy(xws_hbm.at[sidx[b]], rows[b],
                                  gsem[b]).wait()

        def fire_scatter(b):
            pltpu.async_copy(rows[b], acc_sh.at[didx[b]], ssem[b], add=True)

        def wait_scatter(b):
            pltpu.make_async_copy(rows[b], acc_sh.at[didx[b]],
                                  ssem[b]).wait()

        @pl.loop(0, CH)
        def _(i):
            for j in range(d // 16):
                rows[0][i, pl.ds(j * 16, 16)] = jnp.zeros((16,), jnp.float32)

        zp = [(i * CH, CH) for i in range(7)] + [(560, 64)]
        for i, (r0, sz) in enumerate(zp):
            sem = (gsem + ssem)[i]
            pltpu.async_copy(rows[0].at[pl.ds(0, sz)],
                             acc_sh.at[pl.ds(s * RPT + r0, sz)], sem)

        @pl.when(s == NS - 1)
        def _():
            pltpu.async_copy(rows[0].at[pl.ds(0, 16)],
                             acc_sh.at[pl.ds(NS * RPT, 16)], isem[0])
            pltpu.make_async_copy(rows[0].at[pl.ds(0, 16)],
                                  acc_sh.at[pl.ds(NS * RPT, 16)],
                                  isem[0]).wait()

        for i, (r0, sz) in enumerate(zp):
            sem = (gsem + ssem)[i]
            pltpu.make_async_copy(rows[0].at[pl.ds(0, sz)],
                                  acc_sh.at[pl.ds(s * RPT + r0, sz)],
                                  sem).wait()

        plsc.subcore_barrier()

        for b in range(NB):
            fire_idx(0, b)

        @pl.loop(0, NGRP)
        def _(g):
            for b in range(NB):
                wait_idx(b)
                fire_gather(b)
            for b in range(NB):
                wait_gather(b)
                fire_scatter(b)
            for b in range(NB):
                def refill(b=b):
                    wait_scatter(b)
                    fire_idx(g + 1, b)
                pl.when(g < NGRP - 1)(refill)

        wait_scatter(0)
        fire_idx(NGRP, 0)
        wait_idx(0)
        fire_gather(0)
        wait_gather(0)
        fire_scatter(0)
        wait_scatter(0)
        for b in range(1, NB):
            wait_scatter(b)

        plsc.subcore_barrier()

        wp = [(i * CH, CH) for i in range(7)] + [(560, 64)]

        def fire_load(i):
            r0, sz = wp[i]
            pltpu.async_copy(acc_sh.at[pl.ds(s * RPT + r0, sz)],
                             rows[i % NB].at[pl.ds(0, sz)], gsem[i % NB])

        def wait_load(i):
            r0, sz = wp[i]
            pltpu.make_async_copy(acc_sh.at[pl.ds(s * RPT + r0, sz)],
                                  rows[i % NB].at[pl.ds(0, sz)],
                                  gsem[i % NB]).wait()

        def fire_store(i):
            r0, sz = wp[i]
            pltpu.async_copy(rows[i % NB].at[pl.ds(0, sz)],
                             out_hbm.at[c, pl.ds(s * RPT + r0, sz)],
                             ssem[i % NB])

        def wait_store(i):
            r0, sz = wp[i]
            pltpu.make_async_copy(rows[i % NB].at[pl.ds(0, sz)],
                                  out_hbm.at[c, pl.ds(s * RPT + r0, sz)],
                                  ssem[i % NB]).wait()

        for i in range(NB):
            fire_load(i)
        for i in range(len(wp)):
            wait_load(i)
            fire_store(i)
            if i + NB < len(wp):
                wait_store(i)
                fire_load(i + NB)
        for i in range(len(wp) - NB, len(wp)):
            wait_store(i)

        @pl.when(s == NS - 1)
        def _():
            pltpu.sync_copy(acc_sh.at[pl.ds(NS * RPT, 16)],
                            rows[0].at[pl.ds(0, 16)])
            pltpu.sync_copy(rows[0].at[pl.ds(0, 16)],
                            out_hbm.at[c, pl.ds(NS * RPT, 16)])

    return k(src, dst, xws)


_R = 1000


def _tc_mm1(x, w1):
    def body(x_ref, w_ref, out_ref):
        out_ref[...] = jnp.dot(x_ref[...], w_ref[...],
                               preferred_element_type=jnp.float32)

    return pl.pallas_call(
        body,
        grid=(N // _R,),
        in_specs=[
            pl.BlockSpec((_R, 128), lambda i: (i, 0)),
            pl.BlockSpec((128, 128), lambda i: (0, 0)),
        ],
        out_specs=pl.BlockSpec((_R, 128), lambda i: (i, 0)),
        out_shape=jax.ShapeDtypeStruct((N, 128), jnp.float32),
    )(x, w1)


def _tc_scale(degp_t, xw1):
    def body(degp_ref, xw_ref, dinv_ref, xws_ref):
        deg = degp_ref[:, 0:1] + degp_ref[:, 1:2] + 1.0
        dinv = lax.rsqrt(deg)
        dinv_ref[...] = dinv
        xws_ref[...] = xw_ref[...] * dinv

    return pl.pallas_call(
        body,
        grid=(N // _R,),
        in_specs=[
            pl.BlockSpec((_R, NC), lambda i: (i, 0)),
            pl.BlockSpec((_R, 128), lambda i: (i, 0)),
        ],
        out_specs=[
            pl.BlockSpec((_R, 1), lambda i: (i, 0)),
            pl.BlockSpec((_R, 128), lambda i: (i, 0)),
        ],
        out_shape=[
            jax.ShapeDtypeStruct((N, 1), jnp.float32),
            jax.ShapeDtypeStruct((N, 128), jnp.float32),
        ],
    )(degp_t, xw1)


def _tc_mid(acc1, xw1s, dinv, b1, gamma, beta, w2):
    def body(acc_ref, xws_ref, dinv_ref, b_ref, g_ref, be_ref, w_ref, out_ref):
        dinv = dinv_ref[...]
        h = (acc_ref[0] + acc_ref[1] + xws_ref[...]) * dinv + b_ref[...]
        mu = jnp.mean(h, axis=-1, keepdims=True)
        var = jnp.mean((h - mu) ** 2, axis=-1, keepdims=True)
        h = (h - mu) / jnp.sqrt(var + 1e-5) * g_ref[...] + be_ref[...]
        h = jnp.maximum(h, 0.0)
        res = jnp.dot(h, w_ref[...],
                      preferred_element_type=jnp.float32) * dinv
        out_ref[...] = jnp.concatenate(
            [res, jnp.zeros((_R, 64), jnp.float32)], axis=1)

    return pl.pallas_call(
        body,
        grid=(N // _R,),
        in_specs=[
            pl.BlockSpec((NC, _R, 128), lambda i: (0, i, 0)),
            pl.BlockSpec((_R, 128), lambda i: (i, 0)),
            pl.BlockSpec((_R, 1), lambda i: (i, 0)),
            pl.BlockSpec((1, 128), lambda i: (0, 0)),
            pl.BlockSpec((1, 128), lambda i: (0, 0)),
            pl.BlockSpec((1, 128), lambda i: (0, 0)),
            pl.BlockSpec((128, 64), lambda i: (0, 0)),
        ],
        out_specs=pl.BlockSpec((_R, 128), lambda i: (i, 0)),
        out_shape=jax.ShapeDtypeStruct((N, 128), jnp.float32),
    )(acc1, xw1s, dinv, b1, gamma, beta, w2)


def _tc_feat(acc2, xw2s, dinv, b2, wdec):
    def body(acc_ref, xws_ref, dinv_ref, b_ref, wdec_ref, feat_ref, xw3_ref):
        dinv = dinv_ref[...]
        feat = ((acc_ref[0, :, :64] + acc_ref[1, :, :64] + xws_ref[:, :64])
                * dinv + b_ref[...])
        feat_ref[...] = feat
        xw3_ref[...] = jnp.dot(feat, wdec_ref[...],
                               preferred_element_type=jnp.float32) * dinv

    return pl.pallas_call(
        body,
        grid=(N // _R,),
        in_specs=[
            pl.BlockSpec((NC, _R, 128), lambda i: (0, i, 0)),
            pl.BlockSpec((_R, 128), lambda i: (i, 0)),
            pl.BlockSpec((_R, 1), lambda i: (i, 0)),
            pl.BlockSpec((1, 64), lambda i: (0, 0)),
            pl.BlockSpec((64, 128), lambda i: (0, 0)),
        ],
        out_specs=[
            pl.BlockSpec((_R, 64), lambda i: (i, 0)),
            pl.BlockSpec((_R, 128), lambda i: (i, 0)),
        ],
        out_shape=[
            jax.ShapeDtypeStruct((N, 64), jnp.float32),
            jax.ShapeDtypeStruct((N, 128), jnp.float32),
        ],
    )(acc2, xw2s, dinv, b2, wdec)


def _tc_margin(feat, wlab, wdom):
    def body(feat_ref, wlab_ref, wdom_ref, lab_ref, dom_ref):
        feat = feat_ref[...]
        fn = feat / jnp.maximum(
            jnp.sqrt(jnp.sum(feat * feat, axis=1, keepdims=True)), 1e-12)
        wlab = wlab_ref[...]
        wlab = wlab / jnp.maximum(
            jnp.sqrt(jnp.sum(wlab * wlab, axis=0, keepdims=True)), 1e-12)
        lab_ref[...] = 5.0 * jnp.dot(fn, wlab,
                                     preferred_element_type=jnp.float32)
        wdom = wdom_ref[...]
        wdom = wdom / jnp.maximum(
            jnp.sqrt(jnp.sum(wdom * wdom, axis=0, keepdims=True)), 1e-12)
        dom_ref[...] = 5.0 * jnp.dot(fn, wdom,
                                     preferred_element_type=jnp.float32)

    return pl.pallas_call(
        body,
        grid=(N // _R,),
        in_specs=[
            pl.BlockSpec((_R, 64), lambda i: (i, 0)),
            pl.BlockSpec((64, 64), lambda i: (0, 0)),
            pl.BlockSpec((64, 4), lambda i: (0, 0)),
        ],
        out_specs=[
            pl.BlockSpec((_R, 64), lambda i: (i, 0)),
            pl.BlockSpec((_R, 4), lambda i: (i, 0)),
        ],
        out_shape=[
            jax.ShapeDtypeStruct((N, 64), jnp.float32),
            jax.ShapeDtypeStruct((N, 4), jnp.float32),
        ],
    )(feat, wlab, wdom)


def _tc_final(acc3, xw3s, dinv, bdec):
    def body(acc_ref, xws_ref, dinv_ref, b_ref, out_ref):
        out_ref[...] = ((acc_ref[0] + acc_ref[1] + xws_ref[...])
                        * dinv_ref[...] + b_ref[...])

    return pl.pallas_call(
        body,
        grid=(N // _R,),
        in_specs=[
            pl.BlockSpec((NC, _R, 128), lambda i: (0, i, 0)),
            pl.BlockSpec((_R, 128), lambda i: (i, 0)),
            pl.BlockSpec((_R, 1), lambda i: (i, 0)),
            pl.BlockSpec((1, 128), lambda i: (0, 0)),
        ],
        out_specs=pl.BlockSpec((_R, 128), lambda i: (i, 0)),
        out_shape=jax.ShapeDtypeStruct((N, 128), jnp.float32),
    )(acc3, xw3s, dinv, bdec)


def kernel(x, edge_index, W1, b1, gamma, beta, W2, b2, Wdec, bdec, Wdom, Wlab):
    src = edge_index[0]
    dst = edge_index[1]
    degp = _sc_degree(dst)
    xw1 = _tc_mm1(x, W1)
    dinv, xw1s = _tc_scale(degp.reshape(NC, N).T, xw1)
    acc1 = _sc_agg(src, dst, xw1s)
    xw2s = _tc_mid(acc1, xw1s, dinv, b1.reshape(1, -1), gamma.reshape(1, -1),
                   beta.reshape(1, -1), W2)
    acc2 = _sc_agg(src, dst, xw2s)
    feat, xw3s = _tc_feat(acc2, xw2s, dinv, b2.reshape(1, -1), Wdec)
    acc3 = _sc_agg(src, dst, xw3s)
    label_pred, domain_pred = _tc_margin(feat, Wlab, Wdom)
    recon = _tc_final(acc3, xw3s, dinv, bdec.reshape(1, -1))
    return (feat, domain_pred, recon, label_pred)
